# baseline (speedup 1.0000x reference)
import functools

import jax
import jax.numpy as jnp
from jax.experimental import pallas as pl
from jax.experimental.pallas import tpu as pltpu

_BN = 512


def _body(x_ref, a_ref, ck_ref, cb_ref, dk_ref, db_ref, lk_ref, lb_ref,
          out_ref, h1_ref, sum_ref, max_ref, cnt_ref, *, n_blocks, f_in):
    j = pl.program_id(1)
    bn = a_ref.shape[1]

    @pl.when(j == 0)
    def _init():
        h1_ref[...] = jnp.dot(x_ref[0, :, :f_in], ck_ref[...],
                              preferred_element_type=jnp.float32)

    z = jnp.dot(a_ref[0], h1_ref[...], preferred_element_type=jnp.float32)
    z = z + cb_ref[...]

    m = x_ref[0, pl.ds(j * bn, bn), f_in:f_in + 1] != 0.0
    zsum = jnp.sum(jnp.where(m, z, 0.0), axis=0, keepdims=True)
    zmax = jnp.max(jnp.where(m, z, -1e9), axis=0, keepdims=True)
    cnt = jnp.sum(m.astype(jnp.float32))

    @pl.when(j == 0)
    def _first():
        sum_ref[...] = zsum
        max_ref[...] = zmax
        cnt_ref[0, 0] = cnt

    @pl.when(j > 0)
    def _rest():
        sum_ref[...] = sum_ref[...] + zsum
        max_ref[...] = jnp.maximum(max_ref[...], zmax)
        cnt_ref[0, 0] = cnt_ref[0, 0] + cnt

    @pl.when(j == n_blocks - 1)
    def _final():
        avg = sum_ref[...] / jnp.maximum(cnt_ref[0, 0], 1.0)
        pooled = jnp.concatenate([avg, max_ref[...]], axis=1)
        hid = jnp.dot(pooled, dk_ref[...],
                      preferred_element_type=jnp.float32) + db_ref[...]
        hid = jnp.maximum(hid, 0.0)
        out = jnp.dot(hid, lk_ref[...],
                      preferred_element_type=jnp.float32) + lb_ref[...]
        out_ref[0] = out


@jax.jit
def kernel(x, a, conv1_kernel, conv1_bias, dense1_kernel, dense1_bias,
           last_kernel, last_bias):
    B, N, fp1 = x.shape
    f_in = fp1 - 1
    hdim = conv1_kernel.shape[1]
    n_hidden = dense1_kernel.shape[1]
    n_labels = last_kernel.shape[1]
    bn = _BN
    n_blocks = N // bn

    cb = conv1_bias.reshape(1, hdim)
    db = dense1_bias.reshape(1, n_hidden)
    lb = last_bias.reshape(1, n_labels)

    grid = (B, n_blocks)
    out = pl.pallas_call(
        functools.partial(_body, n_blocks=n_blocks, f_in=f_in),
        grid=grid,
        in_specs=[
            pl.BlockSpec((1, N, fp1), lambda b, j: (b, 0, 0)),
            pl.BlockSpec((1, bn, N), lambda b, j: (b, j, 0)),
            pl.BlockSpec((f_in, hdim), lambda b, j: (0, 0)),
            pl.BlockSpec((1, hdim), lambda b, j: (0, 0)),
            pl.BlockSpec((2 * hdim, n_hidden), lambda b, j: (0, 0)),
            pl.BlockSpec((1, n_hidden), lambda b, j: (0, 0)),
            pl.BlockSpec((n_hidden, n_labels), lambda b, j: (0, 0)),
            pl.BlockSpec((1, n_labels), lambda b, j: (0, 0)),
        ],
        out_specs=pl.BlockSpec((1, 1, n_labels), lambda b, j: (b, 0, 0)),
        out_shape=jax.ShapeDtypeStruct((B, 1, n_labels), jnp.float32),
        scratch_shapes=[
            pltpu.VMEM((N, hdim), jnp.float32),
            pltpu.VMEM((1, hdim), jnp.float32),
            pltpu.VMEM((1, hdim), jnp.float32),
            pltpu.SMEM((1, 1), jnp.float32),
        ],
        compiler_params=pltpu.CompilerParams(
            dimension_semantics=("arbitrary", "arbitrary"),
        ),
    )(x, a, conv1_kernel, cb, dense1_kernel, db, last_kernel, lb)
    return out.reshape(B, n_labels)


# speedup vs baseline: 1.1602x; 1.1602x over previous
import functools

import jax
import jax.numpy as jnp
from jax.experimental import pallas as pl
from jax.experimental.pallas import tpu as pltpu

_BN = 1024


def _body(x_ref, a_ref, ck_ref, cb_ref, dk_ref, db_ref, lk_ref, lb_ref,
          out_ref, h1_ref, sum_ref, max_ref, cnt_ref, *, n_blocks, f_in):
    j = pl.program_id(1)
    bn = a_ref.shape[1]

    @pl.when(j == 0)
    def _init():
        h1_ref[...] = jnp.dot(x_ref[0, :, :f_in], ck_ref[...],
                              preferred_element_type=jnp.float32)

    z = jnp.dot(a_ref[0], h1_ref[...], preferred_element_type=jnp.float32)
    z = z + cb_ref[...]

    m = x_ref[0, pl.ds(j * bn, bn), f_in:f_in + 1] != 0.0
    zsum = jnp.sum(jnp.where(m, z, 0.0), axis=0, keepdims=True)
    zmax = jnp.max(jnp.where(m, z, -1e9), axis=0, keepdims=True)
    cnt = jnp.sum(m.astype(jnp.float32))

    @pl.when(j == 0)
    def _first():
        sum_ref[...] = zsum
        max_ref[...] = zmax
        cnt_ref[0, 0] = cnt

    @pl.when(j > 0)
    def _rest():
        sum_ref[...] = sum_ref[...] + zsum
        max_ref[...] = jnp.maximum(max_ref[...], zmax)
        cnt_ref[0, 0] = cnt_ref[0, 0] + cnt

    @pl.when(j == n_blocks - 1)
    def _final():
        avg = sum_ref[...] / jnp.maximum(cnt_ref[0, 0], 1.0)
        pooled = jnp.concatenate([avg, max_ref[...]], axis=1)
        hid = jnp.dot(pooled, dk_ref[...],
                      preferred_element_type=jnp.float32) + db_ref[...]
        hid = jnp.maximum(hid, 0.0)
        out = jnp.dot(hid, lk_ref[...],
                      preferred_element_type=jnp.float32) + lb_ref[...]
        out_ref[0] = out


@jax.jit
def kernel(x, a, conv1_kernel, conv1_bias, dense1_kernel, dense1_bias,
           last_kernel, last_bias):
    B, N, fp1 = x.shape
    f_in = fp1 - 1
    hdim = conv1_kernel.shape[1]
    n_hidden = dense1_kernel.shape[1]
    n_labels = last_kernel.shape[1]
    bn = _BN
    n_blocks = N // bn

    cb = conv1_bias.reshape(1, hdim)
    db = dense1_bias.reshape(1, n_hidden)
    lb = last_bias.reshape(1, n_labels)

    grid = (B, n_blocks)
    out = pl.pallas_call(
        functools.partial(_body, n_blocks=n_blocks, f_in=f_in),
        grid=grid,
        in_specs=[
            pl.BlockSpec((1, N, fp1), lambda b, j: (b, 0, 0)),
            pl.BlockSpec((1, bn, N), lambda b, j: (b, j, 0)),
            pl.BlockSpec((f_in, hdim), lambda b, j: (0, 0)),
            pl.BlockSpec((1, hdim), lambda b, j: (0, 0)),
            pl.BlockSpec((2 * hdim, n_hidden), lambda b, j: (0, 0)),
            pl.BlockSpec((1, n_hidden), lambda b, j: (0, 0)),
            pl.BlockSpec((n_hidden, n_labels), lambda b, j: (0, 0)),
            pl.BlockSpec((1, n_labels), lambda b, j: (0, 0)),
        ],
        out_specs=pl.BlockSpec((1, 1, n_labels), lambda b, j: (b, 0, 0)),
        out_shape=jax.ShapeDtypeStruct((B, 1, n_labels), jnp.float32),
        scratch_shapes=[
            pltpu.VMEM((N, hdim), jnp.float32),
            pltpu.VMEM((1, hdim), jnp.float32),
            pltpu.VMEM((1, hdim), jnp.float32),
            pltpu.SMEM((1, 1), jnp.float32),
        ],
        compiler_params=pltpu.CompilerParams(
            dimension_semantics=("arbitrary", "arbitrary"),
        ),
    )(x, a, conv1_kernel, cb, dense1_kernel, db, last_kernel, lb)
    return out.reshape(B, n_labels)


# speedup vs baseline: 1.1777x; 1.0151x over previous
import functools

import jax
import jax.numpy as jnp
from jax.experimental import pallas as pl
from jax.experimental.pallas import tpu as pltpu

_BN = 2048


def _body(x_ref, a_ref, ck_ref, cb_ref, dk_ref, db_ref, lk_ref, lb_ref,
          out_ref, h1_ref, sum_ref, max_ref, cnt_ref, *, n_blocks, f_in):
    j = pl.program_id(1)
    bn = a_ref.shape[1]

    @pl.when(j == 0)
    def _init():
        h1_ref[...] = jnp.dot(x_ref[0, :, :f_in], ck_ref[...],
                              preferred_element_type=jnp.float32)

    z = jnp.dot(a_ref[0], h1_ref[...], preferred_element_type=jnp.float32)
    z = z + cb_ref[...]

    m = x_ref[0, pl.ds(j * bn, bn), f_in:f_in + 1] != 0.0
    zsum = jnp.sum(jnp.where(m, z, 0.0), axis=0, keepdims=True)
    zmax = jnp.max(jnp.where(m, z, -1e9), axis=0, keepdims=True)
    cnt = jnp.sum(m.astype(jnp.float32))

    @pl.when(j == 0)
    def _first():
        sum_ref[...] = zsum
        max_ref[...] = zmax
        cnt_ref[0, 0] = cnt

    @pl.when(j > 0)
    def _rest():
        sum_ref[...] = sum_ref[...] + zsum
        max_ref[...] = jnp.maximum(max_ref[...], zmax)
        cnt_ref[0, 0] = cnt_ref[0, 0] + cnt

    @pl.when(j == n_blocks - 1)
    def _final():
        avg = sum_ref[...] / jnp.maximum(cnt_ref[0, 0], 1.0)
        pooled = jnp.concatenate([avg, max_ref[...]], axis=1)
        hid = jnp.dot(pooled, dk_ref[...],
                      preferred_element_type=jnp.float32) + db_ref[...]
        hid = jnp.maximum(hid, 0.0)
        out = jnp.dot(hid, lk_ref[...],
                      preferred_element_type=jnp.float32) + lb_ref[...]
        out_ref[0] = out


@jax.jit
def kernel(x, a, conv1_kernel, conv1_bias, dense1_kernel, dense1_bias,
           last_kernel, last_bias):
    B, N, fp1 = x.shape
    f_in = fp1 - 1
    hdim = conv1_kernel.shape[1]
    n_hidden = dense1_kernel.shape[1]
    n_labels = last_kernel.shape[1]
    bn = _BN
    n_blocks = N // bn

    cb = conv1_bias.reshape(1, hdim)
    db = dense1_bias.reshape(1, n_hidden)
    lb = last_bias.reshape(1, n_labels)

    grid = (B, n_blocks)
    out = pl.pallas_call(
        functools.partial(_body, n_blocks=n_blocks, f_in=f_in),
        grid=grid,
        in_specs=[
            pl.BlockSpec((1, N, fp1), lambda b, j: (b, 0, 0)),
            pl.BlockSpec((1, bn, N), lambda b, j: (b, j, 0)),
            pl.BlockSpec((f_in, hdim), lambda b, j: (0, 0)),
            pl.BlockSpec((1, hdim), lambda b, j: (0, 0)),
            pl.BlockSpec((2 * hdim, n_hidden), lambda b, j: (0, 0)),
            pl.BlockSpec((1, n_hidden), lambda b, j: (0, 0)),
            pl.BlockSpec((n_hidden, n_labels), lambda b, j: (0, 0)),
            pl.BlockSpec((1, n_labels), lambda b, j: (0, 0)),
        ],
        out_specs=pl.BlockSpec((1, 1, n_labels), lambda b, j: (b, 0, 0)),
        out_shape=jax.ShapeDtypeStruct((B, 1, n_labels), jnp.float32),
        scratch_shapes=[
            pltpu.VMEM((N, hdim), jnp.float32),
            pltpu.VMEM((1, hdim), jnp.float32),
            pltpu.VMEM((1, hdim), jnp.float32),
            pltpu.SMEM((1, 1), jnp.float32),
        ],
        compiler_params=pltpu.CompilerParams(
            dimension_semantics=("arbitrary", "arbitrary"),
        ),
    )(x, a, conv1_kernel, cb, dense1_kernel, db, last_kernel, lb)
    return out.reshape(B, n_labels)
